# baseline (speedup 1.0000x reference)
import functools

import jax
import jax.numpy as jnp
from jax import lax
from jax.experimental import pallas as pl
from jax.experimental.pallas import tpu as pltpu
from jax.experimental.pallas import tpu_sc as plsc

NUM_CORES = 2
NUM_SUBCORES = 16
NUM_WORKERS = NUM_CORES * NUM_SUBCORES
LANES = 16

BATCH = 16384
EMBED_DIM = 32
B_PER_W = BATCH // NUM_WORKERS
CHUNK = 128
N_CHUNKS = B_PER_W // CHUNK


def _mf_body(uid_hbm, iid_hbm, utab_hbm, itab_hbm, bias_hbm, out_hbm,
             uidx, iidx, urows, irows, bias_v, out_v, sem):
    wid = lax.axis_index("s") * NUM_CORES + lax.axis_index("c")

    base4 = wid * N_CHUNKS
    pltpu.sync_copy(uid_hbm.at[pl.ds(base4, N_CHUNKS)], uidx)
    pltpu.sync_copy(iid_hbm.at[pl.ds(base4, N_CHUNKS)], iidx)

    copies = []
    for j in range(N_CHUNKS):
        dst = pl.ds(j * CHUNK, CHUNK)
        copies.append(pltpu.async_copy(utab_hbm.at[uidx.at[j]],
                                       urows.at[dst], sem))
        copies.append(pltpu.async_copy(itab_hbm.at[iidx.at[j]],
                                       irows.at[dst], sem))
        copies.append(pltpu.async_copy(bias_hbm.at[iidx.at[j]],
                                       bias_v.at[dst], sem))
    for cp in copies:
        cp.wait()

    @pl.loop(0, B_PER_W // LANES)
    def _row_block(i):
        row0 = pl.multiple_of(i * LANES, LANES)
        ridx = row0 + lax.iota(jnp.int32, LANES)
        acc = bias_v[pl.ds(row0, LANES)]
        for d in range(EMBED_DIM):
            cd = jnp.full((LANES,), d, jnp.int32)
            acc = acc + (plsc.load_gather(urows, [ridx, cd])
                         * plsc.load_gather(irows, [ridx, cd]))
        out_v[pl.ds(row0, LANES)] = acc

    pltpu.sync_copy(out_v, out_hbm.at[pl.ds(wid * B_PER_W, B_PER_W)])


_mf_call = functools.partial(
    pl.kernel,
    out_type=jax.ShapeDtypeStruct((BATCH,), jnp.float32),
    mesh=plsc.VectorSubcoreMesh(core_axis_name="c", subcore_axis_name="s",
                                num_cores=NUM_CORES,
                                num_subcores=NUM_SUBCORES),
    scratch_types=[
        pltpu.VMEM((N_CHUNKS, CHUNK), jnp.int32),
        pltpu.VMEM((N_CHUNKS, CHUNK), jnp.int32),
        pltpu.VMEM((B_PER_W, EMBED_DIM), jnp.float32),
        pltpu.VMEM((B_PER_W, EMBED_DIM), jnp.float32),
        pltpu.VMEM((B_PER_W,), jnp.float32),
        pltpu.VMEM((B_PER_W,), jnp.float32),
        pltpu.SemaphoreType.DMA,
    ],
    compiler_params=pltpu.CompilerParams(needs_layout_passes=False,
                                         use_tc_tiling_on_sc=False),
)(_mf_body)


@jax.jit
def kernel(user_ids, item_ids, user_table, item_table, item_bias_table):
    uid = user_ids.astype(jnp.int32).reshape(NUM_WORKERS * N_CHUNKS, CHUNK)
    iid = item_ids.astype(jnp.int32).reshape(NUM_WORKERS * N_CHUNKS, CHUNK)
    bias = item_bias_table.reshape(-1)
    return _mf_call(uid, iid, user_table, item_table, bias)
